# baseline (speedup 1.0000x reference)
import functools

import jax
import jax.numpy as jnp
from jax import lax
from jax.experimental import pallas as pl
from jax.experimental.pallas import tpu as pltpu
from jax.experimental.pallas import tpu_sc as plsc

F32 = jnp.float32
I32 = jnp.int32

_NC = 2
_NS = 16
_NW = _NC * _NS
_L = 16

_C = 80

_mesh = plsc.VectorSubcoreMesh(core_axis_name="c", subcore_axis_name="s")
_sc_params = pltpu.CompilerParams(needs_layout_passes=False)


def _silu(x):
    return x * jax.nn.sigmoid(x)


def _prep_body(h_ref, hi_ref, w1d_ref, w1s_ref, be1_ref, hs_ref, td_ref, ts_ref):
    hs = h_ref[...] + hi_ref[...]
    hs_ref[...] = hs
    td_ref[...] = jnp.dot(hs, w1d_ref[...], preferred_element_type=F32) + be1_ref[...]
    ts_ref[...] = jnp.dot(hs, w1s_ref[...], preferred_element_type=F32)


def _make_gather(N, E, K):
    TPW = K * _C
    NSLOT = 4

    @functools.partial(
        pl.kernel, mesh=_mesh, compiler_params=_sc_params,
        out_type=(jax.ShapeDtypeStruct((E, 128), F32),
                  jax.ShapeDtypeStruct((E,), F32),
                  tuple(jax.ShapeDtypeStruct((E,), F32) for _ in range(3))),
        scratch_types=[
            pltpu.VMEM((TPW,), I32),
            pltpu.VMEM((TPW,), I32),
            [pltpu.VMEM((_C, 128), F32) for _ in range(NSLOT)],
            [pltpu.VMEM((_C,), F32) for _ in range(NSLOT)],
            [[pltpu.VMEM((_C,), F32) for _ in range(3)] for _ in range(NSLOT)],
            pltpu.VMEM((3 * N,), F32),
            [pltpu.SemaphoreType.DMA for _ in range(NSLOT)],
            [pltpu.SemaphoreType.DMA for _ in range(NSLOT)],
        ])
    def gather_k(td_hbm, ts_hbm, posf_hbm, src_hbm, dst_hbm,
                 g_out, d2_out, rel_outs,
                 idx_s, idx_d, bufg, d2buf, relbufs, posv, semg, semw):
        wid = lax.axis_index("s") * _NC + lax.axis_index("c")
        base = wid * TPW
        pltpu.sync_copy(posf_hbm, posv)
        pltpu.sync_copy(dst_hbm.at[pl.ds(base, TPW)], idx_d)
        pltpu.sync_copy(src_hbm.at[pl.ds(base, TPW)], idx_s)

        def start_gd(r, j):
            pltpu.async_copy(td_hbm.at[idx_d.at[pl.ds(j * _C, _C)]],
                             bufg[r], semg[r])

        def start_gs_add(r, j):
            pltpu.async_copy(ts_hbm.at[idx_s.at[pl.ds(j * _C, _C)]],
                             bufg[r], semg[r], add=True)

        def wait_g(r):
            pltpu.make_async_copy(td_hbm.at[pl.ds(0, _C)], bufg[r], semg[r]).wait()

        def geom(r, j):
            for g in range(_C // _L):
                sl = pl.ds(j * _C + _L * g, _L)
                dstv = idx_d[sl]
                srcv = idx_s[sl]
                d2 = jnp.zeros((_L,), F32)
                for a in range(3):
                    pd = plsc.load_gather(posv, [dstv * 3 + a])
                    ps = plsc.load_gather(posv, [srcv * 3 + a])
                    rr = pd - ps
                    relbufs[r][a][pl.ds(_L * g, _L)] = rr
                    d2 = d2 + rr * rr
                d2buf[r][pl.ds(_L * g, _L)] = d2

        def start_write(r, j):
            row0 = base + j * _C
            pltpu.async_copy(bufg[r], g_out.at[pl.ds(row0, _C)], semw[r])
            pltpu.async_copy(d2buf[r], d2_out.at[pl.ds(row0, _C)], semw[r])
            for a in range(3):
                pltpu.async_copy(relbufs[r][a], rel_outs[a].at[pl.ds(row0, _C)],
                                 semw[r])

        def drain_write(r):
            pltpu.make_async_copy(bufg[r], g_out.at[pl.ds(0, _C)], semw[r]).wait()
            pltpu.make_async_copy(d2buf[r], d2_out.at[pl.ds(0, _C)], semw[r]).wait()
            for a in range(3):
                pltpu.make_async_copy(relbufs[r][a], rel_outs[a].at[pl.ds(0, _C)],
                                      semw[r]).wait()

        FB = (K // NSLOT) * NSLOT

        @pl.loop(0, FB, step=NSLOT)
        def _blk(j):
            for r in range(NSLOT):
                @pl.when(j + r >= NSLOT)
                def _(r=r):
                    drain_write(r)
                start_gd(r, j + r)
            for r in range(NSLOT):
                wait_g(r)
                start_gs_add(r, j + r)
                geom(r, j + r)
            for r in range(NSLOT):
                wait_g(r)
                start_write(r, j + r)

        for jj in range(FB, K):
            r = jj % NSLOT
            drain_write(r)
            start_gd(r, jj)
        for jj in range(FB, K):
            r = jj % NSLOT
            wait_g(r)
            start_gs_add(r, jj)
            geom(r, jj)
        for jj in range(FB, K):
            r = jj % NSLOT
            wait_g(r)
            start_write(r, jj)
        for jj in range(K - NSLOT, K):
            drain_write(jj % NSLOT)

    return gather_k


def _edge_body(g_ref, d2_ref, we2_ref, be2_ref, wc1_ref, bc1_ref,
               wc2_ref, wd_ref, m_ref, cw_ref):
    d2row = d2_ref[...]
    ones11 = jnp.ones((1, 1), F32)
    d2col = lax.dot_general(d2row, ones11, (((0,), (0,)), ((), ())),
                            preferred_element_type=F32)
    pre = g_ref[...] + d2col * wd_ref[...]
    m1 = _silu(pre)
    m = _silu(jnp.dot(m1, we2_ref[...], preferred_element_type=F32) + be2_ref[...])
    u = _silu(jnp.dot(m, wc1_ref[...], preferred_element_type=F32) + bc1_ref[...])
    cwrow = lax.dot_general(wc2_ref[...], u, (((0,), (1,)), ((), ())),
                            preferred_element_type=F32)
    m_ref[...] = m
    cw_ref[...] = cwrow


def _make_scatter(N, NP, E, K):
    ZT = 4 * NP // _NS

    @functools.partial(
        pl.kernel, mesh=_mesh, compiler_params=_sc_params,
        out_type=(jax.ShapeDtypeStruct((_NC, NP, 128), F32),
                  jax.ShapeDtypeStruct((_NC, 4 * NP), F32)),
        scratch_types=[
            pltpu.VMEM_SHARED((NP, 128), F32),
            pltpu.VMEM_SHARED((4 * NP,), F32),
            pltpu.VMEM((K, _C), I32),
            [pltpu.VMEM((_C, 128), F32) for _ in range(2)],
            pltpu.VMEM((8, 128), F32),
            pltpu.VMEM((ZT,), F32),
            [pltpu.VMEM((_C,), F32) for _ in range(2)],
            [[pltpu.VMEM((_C,), F32) for _ in range(3)] for _ in range(2)],
            [pltpu.VMEM((_C,), I32) for _ in range(4)],
            [pltpu.VMEM((_C,), F32) for _ in range(4)],
            [pltpu.SemaphoreType.DMA for _ in range(2)],
        ])
    def scatter_k(m_hbm, cw_hbm, relx_hbm, rely_hbm, relz_hbm, dst3_hbm,
                  magg_out, sagg_out,
                  am, asv, idx_d, bufm, zb, zf, cwb, relbs, ibufs, vbufs, seml):
        cid = lax.axis_index("c")
        sid = lax.axis_index("s")
        wid = sid * _NC + cid
        base = wid * (K * _C)
        rel_hbms = (relx_hbm, rely_hbm, relz_hbm)

        def zrow(i, carry):
            for c8 in range(8):
                zb[i, pl.ds(16 * c8, 16)] = jnp.zeros((16,), F32)
            return carry
        lax.fori_loop(0, 8, zrow, 0)

        def zflat(i, carry):
            zf[pl.ds(i * _L, _L)] = jnp.zeros((_L,), F32)
            return carry
        lax.fori_loop(0, ZT // _L, zflat, 0)

        rpt = NP // _NS
        for r in range(rpt // 8):
            pltpu.sync_copy(zb, am.at[pl.ds(sid * rpt + 8 * r, 8)])
        pltpu.sync_copy(zf, asv.at[pl.ds(sid * ZT, ZT)])
        plsc.subcore_barrier()

        pltpu.sync_copy(dst3_hbm.at[wid], idx_d)

        def start_loads(b, j):
            row0 = base + j * _C
            pltpu.async_copy(m_hbm.at[pl.ds(row0, _C)], bufm[b], seml[b])
            for a in range(3):
                pltpu.async_copy(rel_hbms[a].at[pl.ds(row0, _C)],
                                 relbs[b][a], seml[b])
            pltpu.async_copy(cw_hbm.at[pl.ds(row0, _C)], cwb[b], seml[b])

        def wait_loads(b):
            pltpu.make_async_copy(m_hbm.at[pl.ds(0, _C)], bufm[b], seml[b]).wait()
            for a in range(3):
                pltpu.make_async_copy(rel_hbms[a].at[pl.ds(0, _C)],
                                      relbs[b][a], seml[b]).wait()
            pltpu.make_async_copy(cw_hbm.at[pl.ds(0, _C)], cwb[b], seml[b]).wait()

        def accumulate(b, j):
            pltpu.sync_copy(bufm[b], am.at[idx_d.at[j]], add=True)
            for g in range(_C // _L):
                sl = pl.ds(_L * g, _L)
                dstv = idx_d[j, sl]
                cwv = cwb[b][sl]
                for a in range(3):
                    vbufs[a][sl] = relbs[b][a][sl] * cwv
                    ibufs[a][sl] = dstv + (a * NP)
                vbufs[3][sl] = jnp.full((_L,), 1.0, F32)
                ibufs[3][sl] = dstv + (3 * NP)
            for a in range(4):
                pltpu.sync_copy(vbufs[a], asv.at[ibufs[a]], add=True)

        start_loads(0, 0)
        KE = K - 1 if K % 2 else K

        @pl.loop(0, KE, step=2)
        def _pair(j):
            wait_loads(0)
            start_loads(1, j + 1)
            accumulate(0, j)
            wait_loads(1)
            @pl.when(j + 2 < K)
            def _():
                start_loads(0, j + 2)
            accumulate(1, j + 1)

        if K % 2:
            wait_loads(0)
            accumulate(0, K - 1)
        plsc.subcore_barrier()

        for r in range(rpt // 128):
            r0 = sid * rpt + 128 * r
            pltpu.sync_copy(am.at[pl.ds(r0, 128)],
                            magg_out.at[cid].at[pl.ds(r0, 128)])
        pltpu.sync_copy(asv.at[pl.ds(sid * ZT, ZT)],
                        sagg_out.at[cid].at[pl.ds(sid * ZT, ZT)])

    return scatter_k


def _sred_body(*refs):
    out_ref = refs[-1]
    s = None
    for r in refs[:-1]:
        x = r[...]
        part = x[0] + x[1]
        s = part if s is None else s + part
    eye4 = jnp.eye(4, dtype=F32)
    out_ref[...] = lax.dot_general(s, eye4, (((0,), (0,)), ((), ())),
                                   preferred_element_type=F32)


def _node_body(*refs):
    (hs_ref, *mg_refs), (s4_ref, pos_ref, wh1a_ref, wh1b_ref, bh1_ref,
                         wh2_ref, bh2_ref, hout_ref, pout_ref) = (
        refs[:-9], refs[-9:])
    hs = hs_ref[...]
    mg = mg_refs[0][...]
    for r in mg_refs[1:]:
        mg = mg + r[...]
    pre = (jnp.dot(hs, wh1a_ref[...], preferred_element_type=F32)
           + jnp.dot(mg, wh1b_ref[...], preferred_element_type=F32)
           + bh1_ref[...])
    hu = jnp.dot(_silu(pre), wh2_ref[...], preferred_element_type=F32) + bh2_ref[...]
    hout_ref[...] = hs + hu
    s4 = s4_ref[...]
    cnt = jnp.maximum(s4[:, 3:4], 1.0)
    pout_ref[...] = pos_ref[...] + s4[:, 0:3] / cnt


def kernel(h, pos, edge_index, h_init, W_e1, b_e1, W_e2, b_e2, W_c1, b_c1,
           W_c2, W_h1, b_h1, W_h2, b_h2):
    N, D = h.shape
    E = edge_index.shape[1]
    H = W_e2.shape[0]
    NP = 10240
    assert D == 128 and H == 128 and pos.shape[1] == 3
    assert E % (_NW * _C) == 0 and N % 2000 == 0 and N <= NP
    K = E // (_NW * _C)

    srcf = edge_index[0]
    dstf = edge_index[1]
    posf = pos.reshape(-1)
    W1d = W_e1[:128]
    W1s = W_e1[128:256]
    wd = W_e1[256:257]
    Wh1a = W_h1[:128]
    Wh1b = W_h1[128:]
    be1 = b_e1.reshape(1, H)
    be2 = b_e2.reshape(1, H)
    bc1 = b_c1.reshape(1, H)
    bh1 = b_h1.reshape(1, H)
    bh2 = b_h2.reshape(1, D)

    NB = 2000
    nsteps = N // NB
    full = lambda shape: pl.BlockSpec(shape, lambda i: (0, 0))
    rows = lambda w: pl.BlockSpec((NB, w), lambda i: (i, 0))
    hs, td, ts = pl.pallas_call(
        _prep_body,
        grid=(nsteps,),
        in_specs=[rows(128), rows(128), full((128, 128)), full((128, 128)),
                  full((1, 128))],
        out_specs=[rows(128), rows(128), rows(128)],
        out_shape=(jax.ShapeDtypeStruct((N, 128), F32),
                   jax.ShapeDtypeStruct((N, 128), F32),
                   jax.ShapeDtypeStruct((N, 128), F32)),
    )(h, h_init, W1d, W1s, be1)

    BE = 2560
    erows = lambda w: pl.BlockSpec((BE, w), lambda i: (i, 0))
    flat = pl.BlockSpec((1, BE), lambda i: (0, i))

    def edge_mlp(g, d2f, Ei):
        m, cwr = pl.pallas_call(
            _edge_body,
            grid=(Ei // BE,),
            in_specs=[erows(128), flat, full((128, 128)),
                      full((1, 128)), full((128, 128)), full((1, 128)),
                      pl.BlockSpec((128, 1), lambda i: (0, 0)), full((1, 128))],
            out_specs=[erows(128), flat],
            out_shape=(jax.ShapeDtypeStruct((Ei, 128), F32),
                       jax.ShapeDtypeStruct((1, Ei), F32)),
        )(g, d2f.reshape(1, Ei), W_e2, be2, W_c1, bc1, W_c2, wd)
        return m, cwr.reshape(Ei)

    NSL = 3
    kparts = [K // NSL + (1 if i < K % NSL else 0) for i in range(NSL)]
    slices = []
    lo = 0
    for Ki in kparts:
        Ei = _NW * Ki * _C
        src_i = srcf[lo:lo + Ei]
        dst_i = dstf[lo:lo + Ei]
        g, d2f, rels = _make_gather(N, Ei, Ki)(td, ts, posf, src_i, dst_i)
        m, cwf = edge_mlp(g, d2f, Ei)
        slices.append((m, cwf, rels, dst_i.reshape(_NW, Ki, _C), Ki))
        lo += Ei

    aggs = []
    for (m, cwf, rels, dst3_i, Ki) in slices:
        aggs.append(_make_scatter(N, NP, _NW * Ki * _C, Ki)(
            m, cwf, rels[0], rels[1], rels[2], dst3_i))

    BL = 2048
    s4 = pl.pallas_call(
        _sred_body,
        grid=(NP // BL,),
        in_specs=[pl.BlockSpec((_NC, 4, BL), lambda i: (0, 0, i))
                  for _ in range(NSL)],
        out_specs=pl.BlockSpec((BL, 4), lambda i: (i, 0)),
        out_shape=jax.ShapeDtypeStruct((NP, 4), F32),
    )(*[sagg.reshape(_NC, 4, NP) for (_, sagg) in aggs])

    h_out, pos_out = pl.pallas_call(
        _node_body,
        grid=(nsteps,),
        in_specs=([rows(128)] + [rows(128) for _ in range(2 * NSL)]
                  + [rows(4), pl.BlockSpec((NB, 3), lambda i: (i, 0)),
                     full((128, 128)), full((128, 128)), full((1, 128)),
                     full((128, 128)), full((1, 128))]),
        out_specs=[rows(128), pl.BlockSpec((NB, 3), lambda i: (i, 0))],
        out_shape=(jax.ShapeDtypeStruct((N, 128), F32),
                   jax.ShapeDtypeStruct((N, 3), F32)),
    )(hs, *[mg for (magg, _) in aggs for mg in (magg[0], magg[1])],
      s4, pos, Wh1a, Wh1b, bh1, W_h2, bh2)

    return (h_out, pos_out)


# speedup vs baseline: 1.0488x; 1.0488x over previous
import functools

import jax
import jax.numpy as jnp
from jax import lax
from jax.experimental import pallas as pl
from jax.experimental.pallas import tpu as pltpu
from jax.experimental.pallas import tpu_sc as plsc

F32 = jnp.float32
I32 = jnp.int32

_NC = 2
_NS = 16
_NW = _NC * _NS
_L = 16

_C = 80

_mesh = plsc.VectorSubcoreMesh(core_axis_name="c", subcore_axis_name="s")
_sc_params = pltpu.CompilerParams(needs_layout_passes=False)


def _silu(x):
    return x * jax.nn.sigmoid(x)


def _prep_body(h_ref, hi_ref, w1d_ref, w1s_ref, be1_ref, hs_ref, td_ref, ts_ref):
    hs = h_ref[...] + hi_ref[...]
    hs_ref[...] = hs
    td_ref[...] = jnp.dot(hs, w1d_ref[...], preferred_element_type=F32) + be1_ref[...]
    ts_ref[...] = jnp.dot(hs, w1s_ref[...], preferred_element_type=F32)


def _make_gather(N, E, K):
    TPW = K * _C
    NSLOT = 4

    @functools.partial(
        pl.kernel, mesh=_mesh, compiler_params=_sc_params,
        out_type=(jax.ShapeDtypeStruct((E, 128), F32),
                  jax.ShapeDtypeStruct((E,), F32),
                  tuple(jax.ShapeDtypeStruct((E,), F32) for _ in range(3))),
        scratch_types=[
            pltpu.VMEM((TPW,), I32),
            pltpu.VMEM((TPW,), I32),
            [pltpu.VMEM((_C, 128), F32) for _ in range(NSLOT)],
            [pltpu.VMEM((_C,), F32) for _ in range(NSLOT)],
            [[pltpu.VMEM((_C,), F32) for _ in range(3)] for _ in range(NSLOT)],
            pltpu.VMEM((3 * N,), F32),
            [pltpu.SemaphoreType.DMA for _ in range(NSLOT)],
            [pltpu.SemaphoreType.DMA for _ in range(NSLOT)],
        ])
    def gather_k(td_hbm, ts_hbm, posf_hbm, src_hbm, dst_hbm,
                 g_out, d2_out, rel_outs,
                 idx_s, idx_d, bufg, d2buf, relbufs, posv, semg, semw):
        wid = lax.axis_index("s") * _NC + lax.axis_index("c")
        base = wid * TPW
        pltpu.sync_copy(posf_hbm, posv)
        pltpu.sync_copy(dst_hbm.at[pl.ds(base, TPW)], idx_d)
        pltpu.sync_copy(src_hbm.at[pl.ds(base, TPW)], idx_s)

        def start_gd(r, j):
            pltpu.async_copy(td_hbm.at[idx_d.at[pl.ds(j * _C, _C)]],
                             bufg[r], semg[r])

        def start_gs_add(r, j):
            pltpu.async_copy(ts_hbm.at[idx_s.at[pl.ds(j * _C, _C)]],
                             bufg[r], semg[r], add=True)

        def wait_g(r):
            pltpu.make_async_copy(td_hbm.at[pl.ds(0, _C)], bufg[r], semg[r]).wait()

        def geom(r, j):
            for g in range(_C // _L):
                sl = pl.ds(j * _C + _L * g, _L)
                dstv = idx_d[sl]
                srcv = idx_s[sl]
                d2 = jnp.zeros((_L,), F32)
                for a in range(3):
                    pd = plsc.load_gather(posv, [dstv * 3 + a])
                    ps = plsc.load_gather(posv, [srcv * 3 + a])
                    rr = pd - ps
                    relbufs[r][a][pl.ds(_L * g, _L)] = rr
                    d2 = d2 + rr * rr
                d2buf[r][pl.ds(_L * g, _L)] = d2

        def start_write(r, j):
            row0 = base + j * _C
            pltpu.async_copy(bufg[r], g_out.at[pl.ds(row0, _C)], semw[r])
            pltpu.async_copy(d2buf[r], d2_out.at[pl.ds(row0, _C)], semw[r])
            for a in range(3):
                pltpu.async_copy(relbufs[r][a], rel_outs[a].at[pl.ds(row0, _C)],
                                 semw[r])

        def drain_write(r):
            pltpu.make_async_copy(bufg[r], g_out.at[pl.ds(0, _C)], semw[r]).wait()
            pltpu.make_async_copy(d2buf[r], d2_out.at[pl.ds(0, _C)], semw[r]).wait()
            for a in range(3):
                pltpu.make_async_copy(relbufs[r][a], rel_outs[a].at[pl.ds(0, _C)],
                                      semw[r]).wait()

        FB = (K // NSLOT) * NSLOT

        @pl.loop(0, FB, step=NSLOT)
        def _blk(j):
            for r in range(NSLOT):
                @pl.when(j + r >= NSLOT)
                def _(r=r):
                    drain_write(r)
                start_gd(r, j + r)
            for r in range(NSLOT):
                wait_g(r)
                start_gs_add(r, j + r)
                geom(r, j + r)
            for r in range(NSLOT):
                wait_g(r)
                start_write(r, j + r)

        for jj in range(FB, K):
            r = jj % NSLOT
            drain_write(r)
            start_gd(r, jj)
        for jj in range(FB, K):
            r = jj % NSLOT
            wait_g(r)
            start_gs_add(r, jj)
            geom(r, jj)
        for jj in range(FB, K):
            r = jj % NSLOT
            wait_g(r)
            start_write(r, jj)
        for jj in range(K - NSLOT, K):
            drain_write(jj % NSLOT)

    return gather_k


def _edge_body(g_ref, d2_ref, we2_ref, be2_ref, wc1_ref, bc1_ref,
               wc2_ref, wd_ref, m_ref, cw_ref):
    d2row = d2_ref[...]
    ones11 = jnp.ones((1, 1), F32)
    d2col = lax.dot_general(d2row, ones11, (((0,), (0,)), ((), ())),
                            preferred_element_type=F32)
    pre = g_ref[...] + d2col * wd_ref[...]
    m1 = _silu(pre)
    m = _silu(jnp.dot(m1, we2_ref[...], preferred_element_type=F32) + be2_ref[...])
    u = _silu(jnp.dot(m, wc1_ref[...], preferred_element_type=F32) + bc1_ref[...])
    cwrow = lax.dot_general(wc2_ref[...], u, (((0,), (1,)), ((), ())),
                            preferred_element_type=F32)
    m_ref[...] = m
    cw_ref[...] = cwrow


def _make_scatter(N, NP, E, K):
    ZT = 4 * NP // _NS

    @functools.partial(
        pl.kernel, mesh=_mesh, compiler_params=_sc_params,
        out_type=(jax.ShapeDtypeStruct((_NC, NP, 128), F32),
                  jax.ShapeDtypeStruct((_NC, 4 * NP), F32)),
        scratch_types=[
            pltpu.VMEM_SHARED((NP, 128), F32),
            pltpu.VMEM_SHARED((4 * NP,), F32),
            pltpu.VMEM((K, _C), I32),
            [pltpu.VMEM((_C, 128), F32) for _ in range(2)],
            pltpu.VMEM((8, 128), F32),
            pltpu.VMEM((ZT,), F32),
            [pltpu.VMEM((_C,), F32) for _ in range(2)],
            [[pltpu.VMEM((_C,), F32) for _ in range(3)] for _ in range(2)],
            [pltpu.VMEM((_C,), I32) for _ in range(4)],
            [pltpu.VMEM((_C,), F32) for _ in range(4)],
            [pltpu.SemaphoreType.DMA for _ in range(2)],
        ])
    def scatter_k(m_hbm, cw_hbm, relx_hbm, rely_hbm, relz_hbm, dst3_hbm,
                  magg_out, sagg_out,
                  am, asv, idx_d, bufm, zb, zf, cwb, relbs, ibufs, vbufs, seml):
        cid = lax.axis_index("c")
        sid = lax.axis_index("s")
        wid = sid * _NC + cid
        base = wid * (K * _C)
        rel_hbms = (relx_hbm, rely_hbm, relz_hbm)

        def zrow(i, carry):
            for c8 in range(8):
                zb[i, pl.ds(16 * c8, 16)] = jnp.zeros((16,), F32)
            return carry
        lax.fori_loop(0, 8, zrow, 0)

        def zflat(i, carry):
            zf[pl.ds(i * _L, _L)] = jnp.zeros((_L,), F32)
            return carry
        lax.fori_loop(0, ZT // _L, zflat, 0)

        rpt = NP // _NS
        for r in range(rpt // 8):
            pltpu.sync_copy(zb, am.at[pl.ds(sid * rpt + 8 * r, 8)])
        pltpu.sync_copy(zf, asv.at[pl.ds(sid * ZT, ZT)])
        plsc.subcore_barrier()

        pltpu.sync_copy(dst3_hbm.at[wid], idx_d)

        def start_loads(b, j):
            row0 = base + j * _C
            pltpu.async_copy(m_hbm.at[pl.ds(row0, _C)], bufm[b], seml[b])
            for a in range(3):
                pltpu.async_copy(rel_hbms[a].at[pl.ds(row0, _C)],
                                 relbs[b][a], seml[b])
            pltpu.async_copy(cw_hbm.at[pl.ds(row0, _C)], cwb[b], seml[b])

        def wait_loads(b):
            pltpu.make_async_copy(m_hbm.at[pl.ds(0, _C)], bufm[b], seml[b]).wait()
            for a in range(3):
                pltpu.make_async_copy(rel_hbms[a].at[pl.ds(0, _C)],
                                      relbs[b][a], seml[b]).wait()
            pltpu.make_async_copy(cw_hbm.at[pl.ds(0, _C)], cwb[b], seml[b]).wait()

        def accumulate(b, j):
            pltpu.sync_copy(bufm[b], am.at[idx_d.at[j]], add=True)
            for g in range(_C // _L):
                sl = pl.ds(_L * g, _L)
                dstv = idx_d[j, sl]
                cwv = cwb[b][sl]
                for a in range(3):
                    vbufs[a][sl] = relbs[b][a][sl] * cwv
                    ibufs[a][sl] = dstv + (a * NP)
                vbufs[3][sl] = jnp.full((_L,), 1.0, F32)
                ibufs[3][sl] = dstv + (3 * NP)
            for a in range(4):
                pltpu.sync_copy(vbufs[a], asv.at[ibufs[a]], add=True)

        start_loads(0, 0)
        KE = K - 1 if K % 2 else K

        @pl.loop(0, KE, step=2)
        def _pair(j):
            wait_loads(0)
            start_loads(1, j + 1)
            accumulate(0, j)
            wait_loads(1)
            @pl.when(j + 2 < K)
            def _():
                start_loads(0, j + 2)
            accumulate(1, j + 1)

        if K % 2:
            wait_loads(0)
            accumulate(0, K - 1)
        plsc.subcore_barrier()

        for r in range(rpt // 128):
            r0 = sid * rpt + 128 * r
            pltpu.sync_copy(am.at[pl.ds(r0, 128)],
                            magg_out.at[cid].at[pl.ds(r0, 128)])
        pltpu.sync_copy(asv.at[pl.ds(sid * ZT, ZT)],
                        sagg_out.at[cid].at[pl.ds(sid * ZT, ZT)])

    return scatter_k


def _sred_body(*refs):
    out_ref = refs[-1]
    s = None
    for r in refs[:-1]:
        x = r[...]
        part = x[0] + x[1]
        s = part if s is None else s + part
    eye4 = jnp.eye(4, dtype=F32)
    out_ref[...] = lax.dot_general(s, eye4, (((0,), (0,)), ((), ())),
                                   preferred_element_type=F32)


def _node_body(*refs):
    (hs_ref, *mg_refs), (s4_ref, pos_ref, wh1a_ref, wh1b_ref, bh1_ref,
                         wh2_ref, bh2_ref, hout_ref, pout_ref) = (
        refs[:-9], refs[-9:])
    hs = hs_ref[...]
    mg = mg_refs[0][...]
    for r in mg_refs[1:]:
        mg = mg + r[...]
    pre = (jnp.dot(hs, wh1a_ref[...], preferred_element_type=F32)
           + jnp.dot(mg, wh1b_ref[...], preferred_element_type=F32)
           + bh1_ref[...])
    hu = jnp.dot(_silu(pre), wh2_ref[...], preferred_element_type=F32) + bh2_ref[...]
    hout_ref[...] = hs + hu
    s4 = s4_ref[...]
    cnt = jnp.maximum(s4[:, 3:4], 1.0)
    pout_ref[...] = pos_ref[...] + s4[:, 0:3] / cnt


def kernel(h, pos, edge_index, h_init, W_e1, b_e1, W_e2, b_e2, W_c1, b_c1,
           W_c2, W_h1, b_h1, W_h2, b_h2):
    N, D = h.shape
    E = edge_index.shape[1]
    H = W_e2.shape[0]
    NP = 10240
    assert D == 128 and H == 128 and pos.shape[1] == 3
    assert E % (_NW * _C) == 0 and N % 2000 == 0 and N <= NP
    K = E // (_NW * _C)

    srcf = edge_index[0]
    dstf = edge_index[1]
    posf = pos.reshape(-1)
    W1d = W_e1[:128]
    W1s = W_e1[128:256]
    wd = W_e1[256:257]
    Wh1a = W_h1[:128]
    Wh1b = W_h1[128:]
    be1 = b_e1.reshape(1, H)
    be2 = b_e2.reshape(1, H)
    bc1 = b_c1.reshape(1, H)
    bh1 = b_h1.reshape(1, H)
    bh2 = b_h2.reshape(1, D)

    NB = 2000
    nsteps = N // NB
    full = lambda shape: pl.BlockSpec(shape, lambda i: (0, 0))
    rows = lambda w: pl.BlockSpec((NB, w), lambda i: (i, 0))
    hs, td, ts = pl.pallas_call(
        _prep_body,
        grid=(nsteps,),
        in_specs=[rows(128), rows(128), full((128, 128)), full((128, 128)),
                  full((1, 128))],
        out_specs=[rows(128), rows(128), rows(128)],
        out_shape=(jax.ShapeDtypeStruct((N, 128), F32),
                   jax.ShapeDtypeStruct((N, 128), F32),
                   jax.ShapeDtypeStruct((N, 128), F32)),
    )(h, h_init, W1d, W1s, be1)

    BE = 2560
    erows = lambda w: pl.BlockSpec((BE, w), lambda i: (i, 0))
    flat = pl.BlockSpec((1, BE), lambda i: (0, i))

    def edge_mlp(g, d2f, Ei):
        m, cwr = pl.pallas_call(
            _edge_body,
            grid=(Ei // BE,),
            in_specs=[erows(128), flat, full((128, 128)),
                      full((1, 128)), full((128, 128)), full((1, 128)),
                      pl.BlockSpec((128, 1), lambda i: (0, 0)), full((1, 128))],
            out_specs=[erows(128), flat],
            out_shape=(jax.ShapeDtypeStruct((Ei, 128), F32),
                       jax.ShapeDtypeStruct((1, Ei), F32)),
        )(g, d2f.reshape(1, Ei), W_e2, be2, W_c1, bc1, W_c2, wd)
        return m, cwr.reshape(Ei)

    NSL = 2
    kparts = [K // NSL + (1 if i < K % NSL else 0) for i in range(NSL)]
    slices = []
    lo = 0
    for Ki in kparts:
        Ei = _NW * Ki * _C
        src_i = srcf[lo:lo + Ei]
        dst_i = dstf[lo:lo + Ei]
        g, d2f, rels = _make_gather(N, Ei, Ki)(td, ts, posf, src_i, dst_i)
        m, cwf = edge_mlp(g, d2f, Ei)
        slices.append((m, cwf, rels, dst_i.reshape(_NW, Ki, _C), Ki))
        lo += Ei

    aggs = []
    for (m, cwf, rels, dst3_i, Ki) in slices:
        aggs.append(_make_scatter(N, NP, _NW * Ki * _C, Ki)(
            m, cwf, rels[0], rels[1], rels[2], dst3_i))

    BL = 2048
    s4 = pl.pallas_call(
        _sred_body,
        grid=(NP // BL,),
        in_specs=[pl.BlockSpec((_NC, 4, BL), lambda i: (0, 0, i))
                  for _ in range(NSL)],
        out_specs=pl.BlockSpec((BL, 4), lambda i: (i, 0)),
        out_shape=jax.ShapeDtypeStruct((NP, 4), F32),
    )(*[sagg.reshape(_NC, 4, NP) for (_, sagg) in aggs])

    h_out, pos_out = pl.pallas_call(
        _node_body,
        grid=(nsteps,),
        in_specs=([rows(128)] + [rows(128) for _ in range(2 * NSL)]
                  + [rows(4), pl.BlockSpec((NB, 3), lambda i: (i, 0)),
                     full((128, 128)), full((128, 128)), full((1, 128)),
                     full((128, 128)), full((1, 128))]),
        out_specs=[rows(128), pl.BlockSpec((NB, 3), lambda i: (i, 0))],
        out_shape=(jax.ShapeDtypeStruct((N, 128), F32),
                   jax.ShapeDtypeStruct((N, 3), F32)),
    )(hs, *[mg for (magg, _) in aggs for mg in (magg[0], magg[1])],
      s4, pos, Wh1a, Wh1b, bh1, W_h2, bh2)

    return (h_out, pos_out)
